# baseline (speedup 1.0000x reference)
import jax
import jax.numpy as jnp
from jax import lax
from jax.experimental import pallas as pl
from jax.experimental.pallas import tpu as pltpu
from jax.experimental.pallas import tpu_sc as plsc

B = 16384
F = 26
VOCAB1 = 100001
D = 16

_INFO = plsc.get_sparse_core_info()
NC, NS, L = _INFO.num_cores, _INFO.num_subcores, _INFO.num_lanes
NW = NC * NS
ROWS = F * D
RPW = ROWS // NW
QB = 2048
NQ = B // QB
VPQ = QB // L


UNROLL = 8


def _body(xT_hbm, tab_hbm, out_hbm, row_v, idx_v, out0_v, out1_v,
          row_sem, idx_sem, out0_sem, out1_sem):
    wid = lax.axis_index("s") * NC + lax.axis_index("c")
    out_bufs = (out0_v, out1_v)
    out_sems = (out0_sem, out1_sem)

    t = 0
    for r in range(RPW):
        c = wid * RPW + r
        f = c // D
        row_cp = pltpu.async_copy(tab_hbm.at[c], row_v, row_sem)
        if r == 0:
            pltpu.async_copy(xT_hbm.at[f], idx_v, idx_sem).wait()
        else:
            @pl.when(c % D == 0)
            def _load_idx(f=f):
                pltpu.async_copy(xT_hbm.at[f], idx_v, idx_sem).wait()
        row_cp.wait()
        for q in range(NQ):
            ob = out_bufs[t % 2]
            if t >= 2:
                pltpu.make_async_copy(ob, out_hbm.at[c, pl.ds(0, QB)],
                                      out_sems[t % 2]).wait()

            @plsc.parallel_loop(0, QB, L, unroll=UNROLL)
            def gbody(off, q=q, ob=ob):
                ob[pl.ds(off, L)] = plsc.load_gather(
                    row_v, [idx_v[pl.ds(q * QB + off, L)]])
            pltpu.async_copy(ob, out_hbm.at[c, pl.ds(q * QB, QB)],
                             out_sems[t % 2])
            t += 1

    pltpu.make_async_copy(out_bufs[0], out_hbm.at[0, pl.ds(0, QB)],
                          out_sems[0]).wait()
    pltpu.make_async_copy(out_bufs[1], out_hbm.at[0, pl.ds(0, QB)],
                          out_sems[1]).wait()


_gather = pl.kernel(
    _body,
    out_type=jax.ShapeDtypeStruct((ROWS, B), jnp.float32),
    mesh=plsc.VectorSubcoreMesh(core_axis_name="c", subcore_axis_name="s"),
    scratch_types=[
        pltpu.VMEM((VOCAB1,), jnp.float32),
        pltpu.VMEM((B,), jnp.int32),
        pltpu.VMEM((QB,), jnp.float32),
        pltpu.VMEM((QB,), jnp.float32),
        pltpu.SemaphoreType.DMA,
        pltpu.SemaphoreType.DMA,
        pltpu.SemaphoreType.DMA,
        pltpu.SemaphoreType.DMA,
    ],
    compiler_params=pltpu.CompilerParams(
        use_tc_tiling_on_sc=True, needs_layout_passes=False
    ),
)


@jax.jit
def kernel(x, tables):
    x_t = x.T
    tab2 = jnp.transpose(tables, (0, 2, 1)).reshape(ROWS, VOCAB1)
    out_t = _gather(x_t, tab2)
    return out_t.T


# speedup vs baseline: 1.0282x; 1.0282x over previous
import jax
import jax.numpy as jnp
from jax import lax
from jax.experimental import pallas as pl
from jax.experimental.pallas import tpu as pltpu
from jax.experimental.pallas import tpu_sc as plsc

B = 16384
F = 26
VOCAB1 = 100001
D = 16

_INFO = plsc.get_sparse_core_info()
NC, NS, L = _INFO.num_cores, _INFO.num_subcores, _INFO.num_lanes
NW = NC * NS
ROWS = F * D
RPW = ROWS // NW
QB = 4096
NQ = B // QB
VPQ = QB // L


UNROLL = 4


def _body(xT_hbm, tab_hbm, out_hbm, row_v, idx_v, out0_v, out1_v,
          row_sem, idx_sem, out0_sem, out1_sem):
    wid = lax.axis_index("s") * NC + lax.axis_index("c")
    out_bufs = (out0_v, out1_v)
    out_sems = (out0_sem, out1_sem)

    t = 0
    for r in range(RPW):
        c = wid * RPW + r
        f = c // D
        row_cp = pltpu.async_copy(tab_hbm.at[c], row_v, row_sem)
        if r == 0:
            pltpu.async_copy(xT_hbm.at[f], idx_v, idx_sem).wait()
        else:
            @pl.when(c % D == 0)
            def _load_idx(f=f):
                pltpu.async_copy(xT_hbm.at[f], idx_v, idx_sem).wait()
        row_cp.wait()
        for q in range(NQ):
            ob = out_bufs[t % 2]
            if t >= 2:
                pltpu.make_async_copy(ob, out_hbm.at[c, pl.ds(0, QB)],
                                      out_sems[t % 2]).wait()

            @plsc.parallel_loop(0, QB, L, unroll=UNROLL)
            def gbody(off, q=q, ob=ob):
                ob[pl.ds(off, L)] = plsc.load_gather(
                    row_v, [idx_v[pl.ds(q * QB + off, L)]])
            pltpu.async_copy(ob, out_hbm.at[c, pl.ds(q * QB, QB)],
                             out_sems[t % 2])
            t += 1

    pltpu.make_async_copy(out_bufs[0], out_hbm.at[0, pl.ds(0, QB)],
                          out_sems[0]).wait()
    pltpu.make_async_copy(out_bufs[1], out_hbm.at[0, pl.ds(0, QB)],
                          out_sems[1]).wait()


_gather = pl.kernel(
    _body,
    out_type=jax.ShapeDtypeStruct((ROWS, B), jnp.float32),
    mesh=plsc.VectorSubcoreMesh(core_axis_name="c", subcore_axis_name="s"),
    scratch_types=[
        pltpu.VMEM((VOCAB1,), jnp.float32),
        pltpu.VMEM((B,), jnp.int32),
        pltpu.VMEM((QB,), jnp.float32),
        pltpu.VMEM((QB,), jnp.float32),
        pltpu.SemaphoreType.DMA,
        pltpu.SemaphoreType.DMA,
        pltpu.SemaphoreType.DMA,
        pltpu.SemaphoreType.DMA,
    ],
    compiler_params=pltpu.CompilerParams(
        use_tc_tiling_on_sc=True, needs_layout_passes=False
    ),
)


@jax.jit
def kernel(x, tables):
    x_t = x.T
    tab2 = jnp.transpose(tables, (0, 2, 1)).reshape(ROWS, VOCAB1)
    out_t = _gather(x_t, tab2)
    return out_t.T


# speedup vs baseline: 1.0383x; 1.0098x over previous
import jax
import jax.numpy as jnp
from jax import lax
from jax.experimental import pallas as pl
from jax.experimental.pallas import tpu as pltpu
from jax.experimental.pallas import tpu_sc as plsc

B = 16384
F = 26
VOCAB1 = 100001
D = 16

_INFO = plsc.get_sparse_core_info()
NC, NS, L = _INFO.num_cores, _INFO.num_subcores, _INFO.num_lanes
NW = NC * NS
ROWS = F * D
RPW = ROWS // NW
QB = 4096
NQ = B // QB


UNROLL = 8


def _body(xT_hbm, tab_hbm, out_hbm, row_v, idx_v, out0_v, out1_v,
          row_sem, idx_sem, out0_sem, out1_sem):
    wid = lax.axis_index("s") * NC + lax.axis_index("c")
    out_bufs = (out0_v, out1_v)
    out_sems = (out0_sem, out1_sem)

    t = 0
    for r in range(RPW):
        c = wid * RPW + r
        f = c // D
        row_cp = pltpu.async_copy(tab_hbm.at[c], row_v, row_sem)
        if r == 0:
            pltpu.async_copy(xT_hbm.at[f], idx_v, idx_sem).wait()
        else:
            @pl.when(c % D == 0)
            def _load_idx(f=f):
                pltpu.async_copy(xT_hbm.at[f], idx_v, idx_sem).wait()
        row_cp.wait()
        for q in range(NQ):
            ob = out_bufs[t % 2]
            if t >= 2:
                pltpu.make_async_copy(ob, out_hbm.at[c, pl.ds(0, QB)],
                                      out_sems[t % 2]).wait()

            @plsc.parallel_loop(0, QB, L, unroll=UNROLL)
            def gbody(off, q=q, ob=ob):
                ob[pl.ds(off, L)] = plsc.load_gather(
                    row_v, [idx_v[pl.ds(q * QB + off, L)]])
            pltpu.async_copy(ob, out_hbm.at[c, pl.ds(q * QB, QB)],
                             out_sems[t % 2])
            t += 1

    pltpu.make_async_copy(out_bufs[0], out_hbm.at[0, pl.ds(0, QB)],
                          out_sems[0]).wait()
    pltpu.make_async_copy(out_bufs[1], out_hbm.at[0, pl.ds(0, QB)],
                          out_sems[1]).wait()


_gather = pl.kernel(
    _body,
    out_type=jax.ShapeDtypeStruct((ROWS, B), jnp.float32),
    mesh=plsc.VectorSubcoreMesh(core_axis_name="c", subcore_axis_name="s"),
    scratch_types=[
        pltpu.VMEM((VOCAB1,), jnp.float32),
        pltpu.VMEM((B,), jnp.int32),
        pltpu.VMEM((QB,), jnp.float32),
        pltpu.VMEM((QB,), jnp.float32),
        pltpu.SemaphoreType.DMA,
        pltpu.SemaphoreType.DMA,
        pltpu.SemaphoreType.DMA,
        pltpu.SemaphoreType.DMA,
    ],
    compiler_params=pltpu.CompilerParams(
        use_tc_tiling_on_sc=True, needs_layout_passes=False
    ),
)


@jax.jit
def kernel(x, tables):
    x_t = x.T
    tab2 = jnp.transpose(tables, (0, 2, 1)).reshape(ROWS, VOCAB1)
    out_t = _gather(x_t, tab2)
    return out_t.T
